# baseline (speedup 1.0000x reference)
import functools

import jax
import jax.numpy as jnp
from jax import lax
from jax.experimental import pallas as pl
from jax.experimental.pallas import tpu as pltpu
from jax.experimental.pallas import tpu_sc as plsc

NC = 2
NS = 16
NW = NC * NS
LANES = 16

CHUNK = 1024
SLAB = 128
NSLAB = CHUNK // SLAB
SEQS = CHUNK // SLAB


def _rsqrt(var):
    bits = lax.bitcast_convert_type(var, jnp.int32)
    y = lax.bitcast_convert_type(
        jnp.int32(0x5F3759DF) - lax.shift_right_logical(bits, 1), jnp.float32)
    for _ in range(3):
        y = y * (1.5 - 0.5 * var * y * y)
    return y


def _ln_body(d_model, per_w, n_chunks, seq_len,
             idx_hbm, tok_hbm, pos_hbm, gb_hbm, out_hbm,
             idx_v, rows_v, pos_v, gb_v, sem):
    wid = lax.axis_index("s") * NC + lax.axis_index("c")
    base = wid * per_w
    pltpu.sync_copy(pos_hbm, pos_v)
    pltpu.sync_copy(gb_hbm, gb_v)
    h = d_model // 2
    g0 = gb_v[pl.ds(0, LANES)]
    g1 = gb_v[pl.ds(LANES, LANES)]
    b0 = gb_v[pl.ds(d_model, LANES)]
    b1 = gb_v[pl.ds(d_model + LANES, LANES)]
    inv_d = 1.0 / d_model

    def chunk_body(c, carry):
        row0 = pl.multiple_of(base + c * CHUNK, CHUNK)
        pltpu.sync_copy(idx_hbm.at[pl.ds(pl.multiple_of(row0 // SLAB, 8), NSLAB)],
                        idx_v)
        copies = [
            pltpu.make_async_copy(
                tok_hbm.at[idx_v.at[j]],
                rows_v.at[pl.ds(j * SLAB, SLAB)],
                sem,
            )
            for j in range(NSLAB)
        ]
        for cp in copies:
            cp.start()
        for cp in copies:
            cp.wait()

        def pos_body(p, carry2):
            p0 = pos_v[p, pl.ds(0, LANES)]
            p1 = pos_v[p, pl.ds(LANES, LANES)]
            for s in range(SEQS):
                r = s * seq_len + p
                x0 = rows_v[r, pl.ds(0, LANES)] + p0
                x1 = rows_v[r, pl.ds(LANES, LANES)] + p1
                t = x0 + x1
                sm = jnp.sum(t)
                u = x0 * x0 + x1 * x1
                qm = jnp.sum(u)
                mean = sm * inv_d
                var = qm * inv_d - mean * mean + 1e-5
                y = _rsqrt(var)
                o0 = (x0 - mean) * y * g0 + b0
                o1 = (x1 - mean) * y * g1 + b1
                rows_v[r, pl.ds(0, LANES)] = o0
                rows_v[r, pl.ds(LANES, LANES)] = o1
            return carry2

        lax.fori_loop(0, seq_len, pos_body, 0)
        pltpu.sync_copy(rows_v, out_hbm.at[pl.ds(row0, CHUNK)])
        return carry

    lax.fori_loop(0, n_chunks, chunk_body, 0)


def kernel(input_ids, token_table, pos_table, gamma, beta):
    b, s = input_ids.shape
    v, d_model = token_table.shape
    n = b * s
    per_w = n // NW
    n_chunks = per_w // CHUNK

    idx2 = input_ids.reshape(n // SLAB, SLAB).astype(jnp.int32)
    gb_flat = jnp.concatenate([gamma, beta])

    run = pl.kernel(
        functools.partial(_ln_body, d_model, per_w, n_chunks, s),
        out_type=jax.ShapeDtypeStruct((n, d_model), jnp.float32),
        mesh=plsc.VectorSubcoreMesh(core_axis_name="c", subcore_axis_name="s"),
        compiler_params=pltpu.CompilerParams(needs_layout_passes=False,
                                             use_tc_tiling_on_sc=False),
        scratch_types=[
            pltpu.VMEM((NSLAB, SLAB), jnp.int32),
            pltpu.VMEM((CHUNK, d_model), jnp.float32),
            pltpu.VMEM((s, d_model), jnp.float32),
            pltpu.VMEM((2 * d_model,), jnp.float32),
            pltpu.SemaphoreType.DMA,
        ],
    )
    out = run(idx2, token_table, pos_table, gb_flat)
    return out.reshape(b, s, d_model)
